# Initial kernel scaffold; baseline (speedup 1.0000x reference)
import jax
import jax.numpy as jnp
from jax.experimental import pallas as pl


def kernel(title_ids, title_embedding):
    raise NotImplementedError("write your pallas kernel here")


# speedup vs baseline: 4.1578x; 4.1578x over previous
import functools

import jax
import jax.numpy as jnp
from jax import lax
from jax.experimental import pallas as pl
from jax.experimental.pallas import tpu as pltpu
from jax.experimental.pallas import tpu_sc as plsc

_EMB = 64
_NC = 2
_NS = 16
_NW = _NC * _NS
_SUB = 128
_IB = 1024
_NBUF = 2
_HALF = _IB // _NBUF


@functools.cache
def _make_gather(n, emb):
    per_w = n // _NW
    nblk = per_w // _IB
    mesh = plsc.VectorSubcoreMesh(core_axis_name="c", subcore_axis_name="s")

    @functools.partial(
        pl.kernel,
        mesh=mesh,
        out_type=jax.ShapeDtypeStruct((n, emb), jnp.float32),
        compiler_params=pltpu.CompilerParams(use_tc_tiling_on_sc=False),
        scratch_types=[
            pltpu.VMEM((_IB,), jnp.int32),
            pltpu.VMEM((_NBUF, _HALF, emb), jnp.float32),
            pltpu.SemaphoreType.DMA,
            pltpu.SemaphoreType.DMA,
            pltpu.SemaphoreType.DMA,
        ],
    )
    def gather(ids_hbm, table_hbm, out_hbm, idx_v, rows_v, gsem, osem0,
               osem1):
        wid = lax.axis_index("s") * _NC + lax.axis_index("c")
        base = wid * per_w
        osems = (osem0, osem1)

        def body(blk, carry):
            blk_base = base + blk * _IB
            pltpu.sync_copy(ids_hbm.at[pl.ds(blk_base, _IB)], idx_v)
            for b in range(_NBUF):
                @pl.when(blk > 0)
                def _():
                    pltpu.make_async_copy(
                        rows_v.at[b],
                        out_hbm.at[pl.ds(base, _HALF)],
                        osems[b],
                    ).wait()

                handles = [
                    pltpu.async_copy(
                        table_hbm.at[idx_v.at[pl.ds(b * _HALF + j * _SUB,
                                                    _SUB)]],
                        rows_v.at[b].at[pl.ds(j * _SUB, _SUB)],
                        gsem,
                    )
                    for j in range(_HALF // _SUB)
                ]
                for h in handles:
                    h.wait()
                pltpu.async_copy(
                    rows_v.at[b],
                    out_hbm.at[pl.ds(blk_base + b * _HALF, _HALF)],
                    osems[b],
                )
            return carry

        lax.fori_loop(0, nblk, body, 0)
        for b in range(_NBUF):
            pltpu.make_async_copy(
                rows_v.at[b],
                out_hbm.at[pl.ds(base, _HALF)],
                osems[b],
            ).wait()

    return gather


def kernel(title_ids, title_embedding):
    b, h = title_ids.shape
    emb = title_embedding.shape[1]
    ids = title_ids.reshape(-1).astype(jnp.int32)
    out = _make_gather(ids.shape[0], emb)(ids, title_embedding)
    return out.reshape(b, h, emb)


# speedup vs baseline: 4.8352x; 1.1629x over previous
import functools

import jax
import jax.numpy as jnp
from jax import lax
from jax.experimental import pallas as pl
from jax.experimental.pallas import tpu as pltpu
from jax.experimental.pallas import tpu_sc as plsc

_LANE = 128
_NC = 2
_NS = 16
_NW = _NC * _NS
_SUB = 128
_IB = 256
_NBUF = 2
_HALF = _IB // _NBUF
_UNROLL = 8


@functools.cache
def _make_gather(n, emb):
    per_w = n // _NW
    nblk = per_w // _IB
    mesh = plsc.VectorSubcoreMesh(core_axis_name="c", subcore_axis_name="s")

    @functools.partial(
        pl.kernel,
        mesh=mesh,
        out_type=jax.ShapeDtypeStruct((n, emb), jnp.float32),
        compiler_params=pltpu.CompilerParams(use_tc_tiling_on_sc=True),
        scratch_types=[
            pltpu.VMEM((_IB,), jnp.int32),
            pltpu.VMEM((_NBUF, _HALF, _LANE), jnp.float32),
            pltpu.VMEM((_NBUF, _HALF, emb), jnp.float32),
            pltpu.SemaphoreType.DMA,
            pltpu.SemaphoreType.DMA,
            pltpu.SemaphoreType.DMA,
        ],
    )
    def gather(ids_hbm, table_hbm, out_hbm, idx_v, rows_v, out_v, gsem,
               osem0, osem1):
        wid = lax.axis_index("s") * _NC + lax.axis_index("c")
        base = wid * per_w
        osems = (osem0, osem1)
        ngroup = emb // 16

        def body(blk, carry):
            blk_base = base + blk * _IB
            pltpu.sync_copy(ids_hbm.at[pl.ds(blk_base, _IB)], idx_v)
            for b in range(_NBUF):
                @pl.when(blk > 0)
                def _():
                    pltpu.make_async_copy(
                        out_v.at[b],
                        out_hbm.at[pl.ds(base, _HALF)],
                        osems[b],
                    ).wait()

                handles = [
                    pltpu.async_copy(
                        table_hbm.at[idx_v.at[pl.ds(b * _HALF + j * _SUB,
                                                    _SUB)]],
                        rows_v.at[b].at[pl.ds(j * _SUB, _SUB)],
                        gsem,
                    )
                    for j in range(_HALF // _SUB)
                ]
                for h in handles:
                    h.wait()

                def compact(r, carry2):
                    for u in range(_UNROLL):
                        rr = r * _UNROLL + u
                        for j in range(ngroup):
                            out_v[b, rr, pl.ds(j * 16, 16)] = (
                                rows_v[b, rr, pl.ds(j * 16, 16)])
                    return carry2

                lax.fori_loop(0, _HALF // _UNROLL, compact, 0)
                pltpu.async_copy(
                    out_v.at[b],
                    out_hbm.at[pl.ds(blk_base + b * _HALF, _HALF)],
                    osems[b],
                )
            return carry

        lax.fori_loop(0, nblk, body, 0)
        for b in range(_NBUF):
            pltpu.make_async_copy(
                out_v.at[b],
                out_hbm.at[pl.ds(base, _HALF)],
                osems[b],
            ).wait()

    return gather


def kernel(title_ids, title_embedding):
    b, h = title_ids.shape
    emb = title_embedding.shape[1]
    ids = title_ids.reshape(-1).astype(jnp.int32)
    table = jnp.pad(title_embedding, ((0, 0), (0, _LANE - emb)))
    out = _make_gather(ids.shape[0], emb)(ids, table)
    return out.reshape(b, h, emb)
